# Initial kernel scaffold; baseline (speedup 1.0000x reference)
import jax
import jax.numpy as jnp
from jax.experimental import pallas as pl


def kernel(pred, bi_target, tpts, points_meta, pck_meta):
    raise NotImplementedError("write your pallas kernel here")


# speedup vs baseline: 17.9608x; 17.9608x over previous
import functools

import jax
import jax.numpy as jnp
import numpy as np
from jax.experimental import pallas as pl
from jax.experimental.pallas import tpu as pltpu

BS = 4
NPARTS = 16
IMG = 64
S = IMG * IMG
NPOS = 8
NNEG = S - NPOS
APTS = 8
P0 = 5
NPTS = P0 + 2 * APTS + 1
N_VALID = float(NPARTS * NPOS * NNEG)


@functools.lru_cache(maxsize=1)
def _sampling_constants():
    r1h = np.zeros((BS, APTS, NPARTS, NPOS), np.float32)
    r2 = np.zeros((BS, APTS, NPARTS), np.int32)
    pmat = np.zeros((BS, NPTS, NPTS), np.float32)
    with jax.ensure_compile_time_eval():
        key = jax.random.key(1234)
        draws = []
        for i in range(BS):
            k1, k2 = jax.random.split(jax.random.fold_in(key, 2 * i))
            draws.append((
                np.asarray(jax.random.randint(k1, (APTS, NPARTS), 0, NPOS)),
                np.asarray(jax.random.randint(k2, (APTS, NPARTS), 0, NNEG)),
                np.asarray(jax.random.permutation(
                    jax.random.fold_in(key, 2 * i + 1), NPTS)),
            ))
    for i in range(BS):
        r1_i, r2_i, perm = draws[i]
        for a in range(APTS):
            for p in range(NPARTS):
                r1h[i, a, p, r1_i[a, p]] = 1.0
        r2[i] = r2_i
        pmat[i, np.arange(NPTS), perm] = 1.0
    return r1h, r2, pmat


def _body(pred_ref, bt_ref, meta_ref, pmat_ref, r1h_ref, r2_ref, pckm_ref,
          o0_ref, o1_ref, o2_ref, cur_s, dsum_s, psum_s, acc_s, sl_s):
    i = pl.program_id(0)
    j = pl.program_id(1)

    @pl.when(jnp.logical_and(i == 0, j == 0))
    def _init():
        dsum_s[...] = jnp.zeros_like(dsum_s)
        psum_s[...] = jnp.zeros_like(psum_s)
        acc_s[0, 0] = 0.0
        sl_s[0, 0] = 0.0

    pmat = pmat_ref[0]

    @pl.when(j == 0)
    def _stage0():
        pred = pred_ref[0]
        bt = bt_ref[0]
        incl = bt
        sh = 1
        while sh < S:
            incl = incl + jnp.concatenate(
                [jnp.zeros((NPARTS, sh), jnp.float32), incl[:, :-sh]], axis=1)
            sh *= 2
        excl = incl - bt
        excl_i = excl.astype(jnp.int32)
        is_one = bt > 0.5

        ridx = jax.lax.broadcasted_iota(jnp.int32, (NPARTS, NPOS, S), 1)
        pmask = is_one[:, None, :] & (excl_i[:, None, :] == ridx)
        posv = jnp.sum(jnp.where(pmask, pred[:, None, :], 0.0), axis=2)

        negv = jnp.zeros((NPARTS, S), jnp.float32)
        for k in range(NPOS + 1):
            contrib = jnp.where((~is_one) & (excl_i == k), pred, 0.0)
            if k:
                contrib = jnp.concatenate(
                    [contrib[:, k:], jnp.zeros((NPARTS, k), jnp.float32)],
                    axis=1)
            negv = negv + contrib

        colid = jax.lax.broadcasted_iota(jnp.int32, (NPARTS, S), 1)
        valid_col = colid < NNEG
        d = posv[:, :, None] - negv[:, None, :]
        cur = jnp.tanh(d * 0.5)
        cur = jnp.where(valid_col[:, None, :], cur, 0.0)
        cur_s[...] = cur

        s2 = jnp.sum(cur * cur)
        rs_c = jnp.sum(cur, axis=2)
        rs_a = jnp.sum(jnp.abs(cur), axis=2)
        cs_p = jnp.sum(jnp.maximum(cur, 0.0), axis=1)

        atot = s2 - 2.0 * jnp.sum(rs_a) + N_VALID
        e_row = 2.0 * (rs_a - rs_c)
        dis1 = atot + jnp.sum(r1h_ref[0] * e_row[None, :, :], axis=(1, 2))

        sel3 = colid[None] == r2_ref[0][:, :, None]
        dis2 = atot + 4.0 * jnp.sum(
            jnp.where(sel3, cs_p[None], 0.0), axis=(1, 2))

        maxpos = jnp.max(posv, axis=1)
        maxneg = jnp.max(jnp.where(valid_col, negv, -1e30), axis=1)
        acc_i = jnp.mean((maxpos > maxneg).astype(jnp.float32))

        dis_other = jnp.concatenate(
            [dis1, dis2, jnp.reshape(atot, (1,))], axis=0)
        dcontrib = jnp.sum(pmat[:, P0:] * dis_other[None, :], axis=1)
        pcontrib = (jnp.sum(pmat[:, 0:P0] * pckm_ref[0, 0][None, :], axis=1)
                    + jnp.sum(pmat[:, P0:P0 + APTS], axis=1)
                    + pmat[:, NPTS - 1] * acc_i)
        dsum_s[0:1, 0:NPTS] = dsum_s[0:1, 0:NPTS] + dcontrib[None, :]
        psum_s[0:1, 0:NPTS] = psum_s[0:1, 0:NPTS] + pcontrib[None, :]
        acc_s[0, 0] = acc_s[0, 0] + acc_i

        @pl.when(i == BS - 1)
        def _last_mean():
            sl_s[0, 0] = jnp.sum(rs_c) / N_VALID

    m = meta_ref[0, 0]
    diff = cur_s[:, :, 0:NNEG] - m
    dmj = jnp.sum(diff * diff)
    kid = jax.lax.broadcasted_iota(jnp.int32, (NPTS, NPTS), 1)
    pcol = jnp.sum(jnp.where(kid == j, pmat, 0.0), axis=1)
    dsum_s[0:1, 0:NPTS] = dsum_s[0:1, 0:NPTS] + (pcol * dmj)[None, :]

    @pl.when(jnp.logical_and(i == BS - 1, j == P0 - 1))
    def _epilogue():
        dv = dsum_s[0:1, 0:NPTS]
        pv = psum_s[0:1, 0:NPTS]
        pck_t = pv / float(BS)
        wei = 1.0 / (jnp.sqrt(dv) + 1e-8)
        num = jnp.sum(wei * pck_t)
        den = jnp.sum(wei)
        o0_ref[0, 0] = -(num / den)
        o1_ref[0, 0] = acc_s[0, 0] / float(BS)
        o2_ref[0, 0] = sl_s[0, 0]


@jax.jit
def _uniloss_fwd(predr, btr, points_meta, pck_meta, r1h, r2, pmat):
    out_shape = [jax.ShapeDtypeStruct((1, 1), jnp.float32)] * 3
    grid = (BS, P0)
    o0, o1, o2 = pl.pallas_call(
        _body,
        grid=grid,
        in_specs=[
            pl.BlockSpec((1, NPARTS, S), lambda i, j: (i, 0, 0)),
            pl.BlockSpec((1, NPARTS, S), lambda i, j: (i, 0, 0)),
            pl.BlockSpec((1, 1, NPARTS, NPOS, NNEG),
                         lambda i, j: (i, j, 0, 0, 0)),
            pl.BlockSpec((1, NPTS, NPTS), lambda i, j: (i, 0, 0)),
            pl.BlockSpec((1, APTS, NPARTS, NPOS), lambda i, j: (i, 0, 0, 0)),
            pl.BlockSpec((1, APTS, NPARTS), lambda i, j: (i, 0, 0)),
            pl.BlockSpec((1, 1, P0), lambda i, j: (i, 0, 0)),
        ],
        out_specs=[
            pl.BlockSpec(memory_space=pltpu.SMEM),
            pl.BlockSpec(memory_space=pltpu.SMEM),
            pl.BlockSpec(memory_space=pltpu.SMEM),
        ],
        scratch_shapes=[
            pltpu.VMEM((NPARTS, NPOS, S), jnp.float32),
            pltpu.VMEM((8, 128), jnp.float32),
            pltpu.VMEM((8, 128), jnp.float32),
            pltpu.SMEM((1, 1), jnp.float32),
            pltpu.SMEM((1, 1), jnp.float32),
        ],
        out_shape=out_shape,
    )(predr, btr, points_meta, pmat, r1h, r2,
      pck_meta.reshape(BS, 1, P0))
    return (jnp.reshape(o0, ()), jnp.reshape(o1, ()), jnp.reshape(o2, ()))


def kernel(pred, bi_target, tpts, points_meta, pck_meta):
    del tpts
    predr = pred.reshape(BS, NPARTS, S).astype(jnp.float32)
    btr = bi_target.reshape(BS, NPARTS, S).astype(jnp.float32)
    r1h, r2, pmat = _sampling_constants()
    return _uniloss_fwd(predr, btr, points_meta, pck_meta, r1h, r2, pmat)


# speedup vs baseline: 19.7475x; 1.0995x over previous
import functools

import jax
import jax.numpy as jnp
import numpy as np
from jax.experimental import pallas as pl
from jax.experimental.pallas import tpu as pltpu

BS = 4
NPARTS = 16
IMG = 64
S = IMG * IMG
NPOS = 8
NNEG = S - NPOS
APTS = 8
P0 = 5
NPTS = P0 + 2 * APTS + 1
N_VALID = float(NPARTS * NPOS * NNEG)


_R1_RAW = [[[6, 6, 3, 7, 6, 1, 0, 6, 2, 4, 6, 5, 3, 4, 1, 4], [1, 4, 6, 3, 2, 6, 4, 4, 0, 0, 4, 0, 3, 1, 6, 1], [2, 4, 5, 4, 6, 4, 4, 3, 4, 4, 0, 4, 2, 0, 3, 2], [1, 6, 1, 6, 4, 2, 7, 4, 7, 5, 7, 4, 3, 2, 5, 0], [1, 1, 7, 2, 5, 1, 7, 6, 3, 4, 5, 0, 3, 1, 6, 7], [0, 0, 1, 1, 1, 2, 4, 5, 3, 2, 5, 6, 4, 4, 2, 5], [3, 2, 3, 2, 0, 2, 3, 6, 2, 6, 5, 7, 0, 0, 6, 6], [6, 3, 3, 3, 6, 4, 7, 7, 2, 5, 5, 5, 3, 6, 6, 6]], [[2, 7, 0, 6, 4, 4, 4, 1, 2, 1, 3, 6, 3, 1, 7, 2], [0, 5, 1, 1, 1, 5, 7, 5, 6, 6, 0, 6, 5, 1, 1, 6], [6, 6, 6, 4, 2, 5, 6, 0, 6, 2, 7, 0, 3, 0, 2, 7], [6, 0, 5, 3, 6, 3, 6, 3, 3, 2, 7, 4, 1, 4, 1, 6], [1, 1, 0, 7, 0, 1, 4, 7, 4, 6, 5, 7, 3, 7, 7, 7], [6, 4, 7, 0, 4, 7, 5, 6, 2, 2, 4, 7, 1, 4, 0, 7], [6, 4, 1, 1, 5, 4, 0, 0, 5, 0, 7, 0, 1, 3, 7, 6], [3, 7, 1, 6, 7, 3, 0, 4, 3, 0, 4, 2, 5, 2, 2, 3]], [[4, 0, 0, 3, 5, 0, 6, 1, 7, 4, 6, 7, 4, 0, 0, 3], [7, 4, 1, 6, 6, 6, 4, 1, 6, 1, 0, 5, 2, 2, 1, 7], [7, 0, 0, 0, 0, 2, 7, 1, 4, 5, 7, 6, 0, 1, 1, 5], [5, 1, 2, 4, 5, 6, 7, 5, 0, 5, 4, 1, 6, 4, 4, 0], [1, 1, 7, 7, 0, 5, 3, 4, 6, 6, 0, 0, 0, 6, 5, 2], [6, 1, 0, 7, 7, 3, 2, 7, 0, 3, 1, 5, 5, 1, 3, 0], [5, 3, 2, 3, 5, 2, 6, 0, 6, 7, 0, 0, 4, 5, 1, 5], [3, 0, 6, 7, 6, 1, 1, 5, 1, 0, 3, 6, 2, 1, 5, 5]], [[7, 3, 0, 2, 1, 0, 7, 5, 1, 2, 0, 4, 3, 1, 2, 4], [1, 0, 5, 6, 6, 4, 5, 1, 7, 4, 4, 4, 6, 6, 1, 4], [1, 6, 7, 7, 3, 4, 6, 7, 7, 4, 7, 6, 6, 2, 2, 7], [6, 6, 3, 0, 2, 2, 0, 1, 5, 7, 2, 0, 7, 7, 0, 7], [4, 4, 6, 0, 3, 1, 6, 3, 5, 4, 4, 5, 6, 0, 3, 5], [4, 6, 5, 0, 1, 0, 0, 6, 6, 6, 7, 3, 2, 3, 6, 3], [1, 3, 2, 5, 5, 4, 0, 2, 2, 7, 5, 0, 3, 5, 7, 3], [3, 1, 3, 0, 3, 1, 7, 5, 6, 4, 0, 3, 2, 5, 0, 7]]]
_R2_RAW = [[[428, 3936, 3838, 3066, 3855, 1687, 2673, 4031, 2839, 4071, 409, 3223, 107, 1367, 1932, 2212], [4010, 2211, 2421, 3302, 2932, 1993, 3205, 987, 346, 2348, 3288, 618, 1903, 3779, 872, 409], [1267, 3506, 1364, 596, 3434, 609, 2378, 2046, 1329, 3017, 3119, 745, 824, 306, 3609, 1170], [4031, 4025, 3028, 2639, 375, 3548, 61, 4060, 2597, 3439, 1672, 337, 829, 183, 252, 2188], [164, 3193, 1565, 2891, 2093, 589, 163, 268, 3286, 885, 2383, 3500, 1141, 180, 3412, 2488], [4069, 3475, 3750, 1877, 1794, 1271, 921, 3395, 1520, 2249, 3941, 1835, 3728, 3761, 838, 2635], [461, 2958, 0, 1782, 161, 3050, 1847, 202, 3421, 4040, 352, 3821, 3775, 2379, 2149, 3488], [1379, 18, 2948, 2127, 798, 1193, 875, 2334, 1392, 3443, 3518, 2871, 2603, 2890, 263, 1064]], [[1275, 1146, 1402, 3588, 3325, 1711, 832, 540, 1516, 1367, 2464, 2208, 1118, 2905, 19, 2965], [4064, 495, 1386, 3330, 2584, 1753, 1430, 1702, 1765, 3629, 229, 3824, 962, 1079, 1414, 264], [1118, 271, 2023, 3522, 743, 1823, 3521, 1206, 1770, 2377, 3433, 2820, 2123, 1131, 178, 584], [3268, 313, 1015, 2435, 587, 400, 3779, 2269, 1589, 221, 349, 3084, 2895, 384, 146, 1131], [2766, 667, 3900, 3793, 3865, 2873, 1616, 763, 3188, 2738, 1549, 1609, 3751, 2349, 257, 697], [2121, 3970, 3978, 1004, 4009, 944, 3521, 2506, 3462, 1620, 3272, 2985, 3828, 1668, 2487, 2563], [1319, 3638, 2702, 2707, 1683, 2153, 2662, 295, 2055, 3521, 2223, 2655, 2818, 2299, 2931, 462], [1132, 621, 2915, 611, 3689, 874, 934, 2825, 110, 3390, 4063, 3537, 2890, 1560, 355, 2127]], [[1243, 3082, 29, 3811, 1513, 3093, 3363, 2157, 1047, 1180, 2976, 2164, 2182, 3150, 1444, 2134], [958, 1333, 1670, 851, 3955, 1111, 3294, 3389, 3546, 940, 2734, 3021, 974, 3409, 2725, 702], [783, 3329, 845, 0, 1022, 2893, 149, 3475, 3138, 687, 131, 3734, 3229, 4079, 343, 404], [439, 1573, 1501, 564, 1529, 2058, 3736, 358, 2717, 3536, 2205, 3802, 3488, 1702, 1557, 2104], [2257, 115, 855, 2034, 3562, 4013, 3737, 2631, 2943, 3590, 305, 2858, 3158, 780, 553, 1715], [1831, 640, 217, 3101, 2964, 323, 3434, 1411, 1701, 1736, 1009, 3985, 4001, 2925, 1802, 69], [2948, 1499, 2356, 923, 1021, 3532, 2685, 1481, 1817, 1921, 1499, 1062, 777, 3565, 746, 358], [988, 2844, 3454, 156, 4046, 1313, 902, 3366, 1236, 3490, 3736, 3500, 3293, 1492, 1953, 2186]], [[3655, 736, 3087, 3365, 1951, 2281, 3498, 1180, 3546, 3466, 3096, 1835, 696, 1529, 2773, 2364], [2761, 283, 1907, 3739, 741, 810, 1359, 760, 1212, 1450, 74, 6, 452, 2992, 2406, 392], [4008, 875, 1232, 2207, 718, 3618, 1585, 4023, 2847, 3051, 3307, 3043, 2066, 582, 1590, 1287], [1854, 1265, 1837, 2291, 2514, 1483, 2030, 2401, 715, 676, 1613, 905, 3707, 4047, 3896, 3150], [2015, 2149, 3580, 2945, 258, 3168, 2914, 2934, 389, 886, 664, 2758, 892, 831, 1498, 150], [568, 4047, 225, 965, 1036, 1130, 1916, 3370, 696, 436, 3666, 2041, 1047, 119, 3280, 3920], [3336, 2221, 2916, 3660, 198, 3786, 3866, 2795, 3101, 2816, 3128, 1606, 1315, 145, 3679, 2620], [2148, 726, 2452, 4057, 2172, 2361, 2865, 2817, 326, 1668, 2553, 169, 2324, 3411, 3601, 2494]]]
_PERM_RAW = [[1, 19, 2, 16, 3, 8, 7, 18, 15, 5, 17, 6, 12, 10, 4, 14, 0, 13, 11, 20, 9, 21], [2, 11, 19, 9, 14, 0, 20, 17, 12, 7, 3, 16, 4, 21, 6, 13, 10, 5, 18, 8, 15, 1], [21, 1, 12, 11, 7, 8, 17, 0, 20, 18, 14, 9, 16, 10, 13, 6, 15, 19, 5, 4, 2, 3], [5, 1, 20, 14, 17, 7, 12, 6, 0, 19, 2, 10, 13, 4, 8, 16, 9, 15, 21, 3, 18, 11]]


@functools.lru_cache(maxsize=1)
def _sampling_constants():
    r1 = np.asarray(_R1_RAW, np.int32)
    r2 = np.asarray(_R2_RAW, np.int32)
    perm = np.asarray(_PERM_RAW, np.int32)
    ib, ia, ip = np.indices((BS, APTS, NPARTS))
    r1h = np.zeros((BS, APTS, NPARTS, NPOS), np.float32)
    r1h[ib, ia, ip, r1] = 1.0
    pmat = np.zeros((BS, NPTS, NPTS), np.float32)
    pmat[np.arange(BS)[:, None], np.arange(NPTS)[None, :], perm] = 1.0
    return r1h, r2, pmat


def _body(pred_ref, bt_ref, meta_ref, pmat_ref, r1h_ref, r2_ref, pckm_ref,
          o0_ref, o1_ref, o2_ref, cur_s, dsum_s, psum_s, acc_s, sl_s):
    i = pl.program_id(0)
    j = pl.program_id(1)

    @pl.when(jnp.logical_and(i == 0, j == 0))
    def _init():
        dsum_s[...] = jnp.zeros_like(dsum_s)
        psum_s[...] = jnp.zeros_like(psum_s)
        acc_s[0, 0] = 0.0
        sl_s[0, 0] = 0.0

    pmat = pmat_ref[0]

    @pl.when(j == 0)
    def _stage0():
        pred = pred_ref[0]
        bt = bt_ref[0]
        incl = bt
        sh = 1
        while sh < S:
            incl = incl + jnp.concatenate(
                [jnp.zeros((NPARTS, sh), jnp.float32), incl[:, :-sh]], axis=1)
            sh *= 2
        excl = incl - bt
        excl_i = excl.astype(jnp.int32)
        is_one = bt > 0.5
        pos_pred = jnp.where(is_one, pred, 0.0)

        ridx = jax.lax.broadcasted_iota(jnp.int32, (NPARTS, NPOS, S), 1)
        pmask = excl_i[:, None, :] == ridx
        posv = jnp.sum(jnp.where(pmask, pos_pred[:, None, :], 0.0), axis=2)

        negv = jnp.zeros((NPARTS, S), jnp.float32)
        for k in range(NPOS + 1):
            contrib = jnp.where((~is_one) & (excl_i == k), pred, 0.0)
            if k:
                contrib = jnp.concatenate(
                    [contrib[:, k:], jnp.zeros((NPARTS, k), jnp.float32)],
                    axis=1)
            negv = negv + contrib

        colid = jax.lax.broadcasted_iota(jnp.int32, (NPARTS, S), 1)
        valid_col = colid < NNEG
        posv_flat = jnp.reshape(posv, (NPARTS * NPOS, 1))
        negv_rep = jnp.reshape(
            jnp.broadcast_to(negv[:, None, :], (NPARTS, NPOS, S)),
            (NPARTS * NPOS, S))
        valid2 = jax.lax.broadcasted_iota(
            jnp.int32, (NPARTS * NPOS, S), 1) < NNEG
        cur = jnp.where(valid2, jnp.tanh((posv_flat - negv_rep) * 0.5), 0.0)
        cur_s[...] = cur

        s2 = jnp.sum(cur * cur)
        rs_flat_c = jnp.sum(cur, axis=1)
        rs_flat_a = jnp.sum(jnp.abs(cur), axis=1)
        ones_sel = (jax.lax.broadcasted_iota(
            jnp.int32, (NPARTS, NPARTS * NPOS), 1) // NPOS
            == jax.lax.broadcasted_iota(
                jnp.int32, (NPARTS, NPARTS * NPOS), 0)).astype(jnp.float32)
        cs_p = jnp.dot(ones_sel, jnp.maximum(cur, 0.0),
                       preferred_element_type=jnp.float32)

        atot = s2 - 2.0 * jnp.sum(rs_flat_a) + N_VALID
        e_row = 2.0 * jnp.reshape(rs_flat_a - rs_flat_c, (NPARTS, NPOS))
        dis1 = atot + jnp.sum(r1h_ref[0] * e_row[None, :, :], axis=(1, 2))

        sel3 = colid[None] == r2_ref[0][:, :, None]
        dis2 = atot + 4.0 * jnp.sum(
            jnp.where(sel3, cs_p[None], 0.0), axis=(1, 2))

        maxpos = jnp.max(posv, axis=1)
        maxneg = jnp.max(jnp.where(valid_col, negv, -1e30), axis=1)
        acc_i = jnp.mean((maxpos > maxneg).astype(jnp.float32))

        dis_other = jnp.concatenate(
            [dis1, dis2, jnp.reshape(atot, (1,))], axis=0)
        dcontrib = jnp.sum(pmat[:, P0:] * dis_other[None, :], axis=1)
        pcontrib = (jnp.sum(pmat[:, 0:P0] * pckm_ref[0, 0][None, :], axis=1)
                    + jnp.sum(pmat[:, P0:P0 + APTS], axis=1)
                    + pmat[:, NPTS - 1] * acc_i)
        dsum_s[0:1, 0:NPTS] = dsum_s[0:1, 0:NPTS] + dcontrib[None, :]
        psum_s[0:1, 0:NPTS] = psum_s[0:1, 0:NPTS] + pcontrib[None, :]
        acc_s[0, 0] = acc_s[0, 0] + acc_i

        @pl.when(i == BS - 1)
        def _last_mean():
            sl_s[0, 0] = jnp.sum(rs_flat_c) / N_VALID

    m = meta_ref[0, 0]
    diff = cur_s[:, 0:NNEG] - m
    dmj = jnp.sum(diff * diff)
    kid = jax.lax.broadcasted_iota(jnp.int32, (NPTS, NPTS), 1)
    pcol = jnp.sum(jnp.where(kid == j, pmat, 0.0), axis=1)
    dsum_s[0:1, 0:NPTS] = dsum_s[0:1, 0:NPTS] + (pcol * dmj)[None, :]

    @pl.when(jnp.logical_and(i == BS - 1, j == P0 - 1))
    def _epilogue():
        dv = dsum_s[0:1, 0:NPTS]
        pv = psum_s[0:1, 0:NPTS]
        pck_t = pv / float(BS)
        wei = 1.0 / (jnp.sqrt(dv) + 1e-8)
        num = jnp.sum(wei * pck_t)
        den = jnp.sum(wei)
        o0_ref[0, 0] = -(num / den)
        o1_ref[0, 0] = acc_s[0, 0] / float(BS)
        o2_ref[0, 0] = sl_s[0, 0]


@jax.jit
def _uniloss_fwd(predr, btr, points_meta, pck_meta, r1h, r2, pmat):
    out_shape = [jax.ShapeDtypeStruct((1, 1), jnp.float32)] * 3
    grid = (BS, P0)
    o0, o1, o2 = pl.pallas_call(
        _body,
        grid=grid,
        in_specs=[
            pl.BlockSpec((1, NPARTS, S), lambda i, j: (i, 0, 0)),
            pl.BlockSpec((1, NPARTS, S), lambda i, j: (i, 0, 0)),
            pl.BlockSpec((1, 1, NPARTS * NPOS, NNEG),
                         lambda i, j: (i, j, 0, 0)),
            pl.BlockSpec((1, NPTS, NPTS), lambda i, j: (i, 0, 0)),
            pl.BlockSpec((1, APTS, NPARTS, NPOS), lambda i, j: (i, 0, 0, 0)),
            pl.BlockSpec((1, APTS, NPARTS), lambda i, j: (i, 0, 0)),
            pl.BlockSpec((1, 1, P0), lambda i, j: (i, 0, 0)),
        ],
        out_specs=[
            pl.BlockSpec(memory_space=pltpu.SMEM),
            pl.BlockSpec(memory_space=pltpu.SMEM),
            pl.BlockSpec(memory_space=pltpu.SMEM),
        ],
        scratch_shapes=[
            pltpu.VMEM((NPARTS * NPOS, S), jnp.float32),
            pltpu.VMEM((8, 128), jnp.float32),
            pltpu.VMEM((8, 128), jnp.float32),
            pltpu.SMEM((1, 1), jnp.float32),
            pltpu.SMEM((1, 1), jnp.float32),
        ],
        out_shape=out_shape,
    )(predr, btr, points_meta.reshape(BS, P0, NPARTS * NPOS, NNEG),
      pmat, r1h, r2, pck_meta.reshape(BS, 1, P0))
    return (jnp.reshape(o0, ()), jnp.reshape(o1, ()), jnp.reshape(o2, ()))


def kernel(pred, bi_target, tpts, points_meta, pck_meta):
    del tpts
    predr = pred.reshape(BS, NPARTS, S).astype(jnp.float32)
    btr = bi_target.reshape(BS, NPARTS, S).astype(jnp.float32)
    r1h, r2, pmat = _sampling_constants()
    return _uniloss_fwd(predr, btr, points_meta, pck_meta, r1h, r2, pmat)


# speedup vs baseline: 30.5358x; 1.5463x over previous
import functools

import jax
import jax.numpy as jnp
import numpy as np
from jax.experimental import pallas as pl
from jax.experimental.pallas import tpu as pltpu

BS = 4
NPARTS = 16
IMG = 64
S = IMG * IMG
NPOS = 8
NNEG = S - NPOS
APTS = 8
P0 = 5
NPTS = P0 + 2 * APTS + 1
N_VALID = float(NPARTS * NPOS * NNEG)


_R1_RAW = [[[6, 6, 3, 7, 6, 1, 0, 6, 2, 4, 6, 5, 3, 4, 1, 4], [1, 4, 6, 3, 2, 6, 4, 4, 0, 0, 4, 0, 3, 1, 6, 1], [2, 4, 5, 4, 6, 4, 4, 3, 4, 4, 0, 4, 2, 0, 3, 2], [1, 6, 1, 6, 4, 2, 7, 4, 7, 5, 7, 4, 3, 2, 5, 0], [1, 1, 7, 2, 5, 1, 7, 6, 3, 4, 5, 0, 3, 1, 6, 7], [0, 0, 1, 1, 1, 2, 4, 5, 3, 2, 5, 6, 4, 4, 2, 5], [3, 2, 3, 2, 0, 2, 3, 6, 2, 6, 5, 7, 0, 0, 6, 6], [6, 3, 3, 3, 6, 4, 7, 7, 2, 5, 5, 5, 3, 6, 6, 6]], [[2, 7, 0, 6, 4, 4, 4, 1, 2, 1, 3, 6, 3, 1, 7, 2], [0, 5, 1, 1, 1, 5, 7, 5, 6, 6, 0, 6, 5, 1, 1, 6], [6, 6, 6, 4, 2, 5, 6, 0, 6, 2, 7, 0, 3, 0, 2, 7], [6, 0, 5, 3, 6, 3, 6, 3, 3, 2, 7, 4, 1, 4, 1, 6], [1, 1, 0, 7, 0, 1, 4, 7, 4, 6, 5, 7, 3, 7, 7, 7], [6, 4, 7, 0, 4, 7, 5, 6, 2, 2, 4, 7, 1, 4, 0, 7], [6, 4, 1, 1, 5, 4, 0, 0, 5, 0, 7, 0, 1, 3, 7, 6], [3, 7, 1, 6, 7, 3, 0, 4, 3, 0, 4, 2, 5, 2, 2, 3]], [[4, 0, 0, 3, 5, 0, 6, 1, 7, 4, 6, 7, 4, 0, 0, 3], [7, 4, 1, 6, 6, 6, 4, 1, 6, 1, 0, 5, 2, 2, 1, 7], [7, 0, 0, 0, 0, 2, 7, 1, 4, 5, 7, 6, 0, 1, 1, 5], [5, 1, 2, 4, 5, 6, 7, 5, 0, 5, 4, 1, 6, 4, 4, 0], [1, 1, 7, 7, 0, 5, 3, 4, 6, 6, 0, 0, 0, 6, 5, 2], [6, 1, 0, 7, 7, 3, 2, 7, 0, 3, 1, 5, 5, 1, 3, 0], [5, 3, 2, 3, 5, 2, 6, 0, 6, 7, 0, 0, 4, 5, 1, 5], [3, 0, 6, 7, 6, 1, 1, 5, 1, 0, 3, 6, 2, 1, 5, 5]], [[7, 3, 0, 2, 1, 0, 7, 5, 1, 2, 0, 4, 3, 1, 2, 4], [1, 0, 5, 6, 6, 4, 5, 1, 7, 4, 4, 4, 6, 6, 1, 4], [1, 6, 7, 7, 3, 4, 6, 7, 7, 4, 7, 6, 6, 2, 2, 7], [6, 6, 3, 0, 2, 2, 0, 1, 5, 7, 2, 0, 7, 7, 0, 7], [4, 4, 6, 0, 3, 1, 6, 3, 5, 4, 4, 5, 6, 0, 3, 5], [4, 6, 5, 0, 1, 0, 0, 6, 6, 6, 7, 3, 2, 3, 6, 3], [1, 3, 2, 5, 5, 4, 0, 2, 2, 7, 5, 0, 3, 5, 7, 3], [3, 1, 3, 0, 3, 1, 7, 5, 6, 4, 0, 3, 2, 5, 0, 7]]]
_R2_RAW = [[[428, 3936, 3838, 3066, 3855, 1687, 2673, 4031, 2839, 4071, 409, 3223, 107, 1367, 1932, 2212], [4010, 2211, 2421, 3302, 2932, 1993, 3205, 987, 346, 2348, 3288, 618, 1903, 3779, 872, 409], [1267, 3506, 1364, 596, 3434, 609, 2378, 2046, 1329, 3017, 3119, 745, 824, 306, 3609, 1170], [4031, 4025, 3028, 2639, 375, 3548, 61, 4060, 2597, 3439, 1672, 337, 829, 183, 252, 2188], [164, 3193, 1565, 2891, 2093, 589, 163, 268, 3286, 885, 2383, 3500, 1141, 180, 3412, 2488], [4069, 3475, 3750, 1877, 1794, 1271, 921, 3395, 1520, 2249, 3941, 1835, 3728, 3761, 838, 2635], [461, 2958, 0, 1782, 161, 3050, 1847, 202, 3421, 4040, 352, 3821, 3775, 2379, 2149, 3488], [1379, 18, 2948, 2127, 798, 1193, 875, 2334, 1392, 3443, 3518, 2871, 2603, 2890, 263, 1064]], [[1275, 1146, 1402, 3588, 3325, 1711, 832, 540, 1516, 1367, 2464, 2208, 1118, 2905, 19, 2965], [4064, 495, 1386, 3330, 2584, 1753, 1430, 1702, 1765, 3629, 229, 3824, 962, 1079, 1414, 264], [1118, 271, 2023, 3522, 743, 1823, 3521, 1206, 1770, 2377, 3433, 2820, 2123, 1131, 178, 584], [3268, 313, 1015, 2435, 587, 400, 3779, 2269, 1589, 221, 349, 3084, 2895, 384, 146, 1131], [2766, 667, 3900, 3793, 3865, 2873, 1616, 763, 3188, 2738, 1549, 1609, 3751, 2349, 257, 697], [2121, 3970, 3978, 1004, 4009, 944, 3521, 2506, 3462, 1620, 3272, 2985, 3828, 1668, 2487, 2563], [1319, 3638, 2702, 2707, 1683, 2153, 2662, 295, 2055, 3521, 2223, 2655, 2818, 2299, 2931, 462], [1132, 621, 2915, 611, 3689, 874, 934, 2825, 110, 3390, 4063, 3537, 2890, 1560, 355, 2127]], [[1243, 3082, 29, 3811, 1513, 3093, 3363, 2157, 1047, 1180, 2976, 2164, 2182, 3150, 1444, 2134], [958, 1333, 1670, 851, 3955, 1111, 3294, 3389, 3546, 940, 2734, 3021, 974, 3409, 2725, 702], [783, 3329, 845, 0, 1022, 2893, 149, 3475, 3138, 687, 131, 3734, 3229, 4079, 343, 404], [439, 1573, 1501, 564, 1529, 2058, 3736, 358, 2717, 3536, 2205, 3802, 3488, 1702, 1557, 2104], [2257, 115, 855, 2034, 3562, 4013, 3737, 2631, 2943, 3590, 305, 2858, 3158, 780, 553, 1715], [1831, 640, 217, 3101, 2964, 323, 3434, 1411, 1701, 1736, 1009, 3985, 4001, 2925, 1802, 69], [2948, 1499, 2356, 923, 1021, 3532, 2685, 1481, 1817, 1921, 1499, 1062, 777, 3565, 746, 358], [988, 2844, 3454, 156, 4046, 1313, 902, 3366, 1236, 3490, 3736, 3500, 3293, 1492, 1953, 2186]], [[3655, 736, 3087, 3365, 1951, 2281, 3498, 1180, 3546, 3466, 3096, 1835, 696, 1529, 2773, 2364], [2761, 283, 1907, 3739, 741, 810, 1359, 760, 1212, 1450, 74, 6, 452, 2992, 2406, 392], [4008, 875, 1232, 2207, 718, 3618, 1585, 4023, 2847, 3051, 3307, 3043, 2066, 582, 1590, 1287], [1854, 1265, 1837, 2291, 2514, 1483, 2030, 2401, 715, 676, 1613, 905, 3707, 4047, 3896, 3150], [2015, 2149, 3580, 2945, 258, 3168, 2914, 2934, 389, 886, 664, 2758, 892, 831, 1498, 150], [568, 4047, 225, 965, 1036, 1130, 1916, 3370, 696, 436, 3666, 2041, 1047, 119, 3280, 3920], [3336, 2221, 2916, 3660, 198, 3786, 3866, 2795, 3101, 2816, 3128, 1606, 1315, 145, 3679, 2620], [2148, 726, 2452, 4057, 2172, 2361, 2865, 2817, 326, 1668, 2553, 169, 2324, 3411, 3601, 2494]]]
_PERM_RAW = [[1, 19, 2, 16, 3, 8, 7, 18, 15, 5, 17, 6, 12, 10, 4, 14, 0, 13, 11, 20, 9, 21], [2, 11, 19, 9, 14, 0, 20, 17, 12, 7, 3, 16, 4, 21, 6, 13, 10, 5, 18, 8, 15, 1], [21, 1, 12, 11, 7, 8, 17, 0, 20, 18, 14, 9, 16, 10, 13, 6, 15, 19, 5, 4, 2, 3], [5, 1, 20, 14, 17, 7, 12, 6, 0, 19, 2, 10, 13, 4, 8, 16, 9, 15, 21, 3, 18, 11]]


@functools.lru_cache(maxsize=1)
def _sampling_constants():
    r1 = np.asarray(_R1_RAW, np.int32)
    r2 = np.asarray(_R2_RAW, np.int32)
    perm = np.asarray(_PERM_RAW, np.int32)
    ib, ia, ip = np.indices((BS, APTS, NPARTS))
    r1h = np.zeros((BS, APTS, NPARTS, NPOS), np.float32)
    r1h[ib, ia, ip, r1] = 1.0
    pmat = np.zeros((BS, NPTS, NPTS), np.float32)
    pmat[np.arange(BS)[:, None], np.arange(NPTS)[None, :], perm] = 1.0
    return r1h, r2, pmat


def _body(pred_ref, bt_ref, meta_ref, pmat_ref, r1h_ref, r2_ref, pckm_ref,
          o0_ref, o1_ref, o2_ref, dsum_s, psum_s, acc_s, sl_s):
    i = pl.program_id(0)

    @pl.when(i == 0)
    def _init():
        dsum_s[...] = jnp.zeros_like(dsum_s)
        psum_s[...] = jnp.zeros_like(psum_s)
        acc_s[0, 0] = 0.0
        sl_s[0, 0] = 0.0

    pmat = pmat_ref[0]

    if True:
        pred = pred_ref[0]
        bt = bt_ref[0]
        incl = bt
        sh = 1
        while sh < S:
            incl = incl + jnp.concatenate(
                [jnp.zeros((NPARTS, sh), jnp.float32), incl[:, :-sh]], axis=1)
            sh *= 2
        excl = incl - bt
        excl_i = excl.astype(jnp.int32)
        is_one = bt > 0.5
        pos_pred = jnp.where(is_one, pred, 0.0)

        ridx = jax.lax.broadcasted_iota(jnp.int32, (NPARTS, NPOS, S), 1)
        pmask = excl_i[:, None, :] == ridx
        posv = jnp.sum(jnp.where(pmask, pos_pred[:, None, :], 0.0), axis=2)

        negv = jnp.zeros((NPARTS, S), jnp.float32)
        for k in range(NPOS + 1):
            contrib = jnp.where((~is_one) & (excl_i == k), pred, 0.0)
            if k:
                contrib = jnp.concatenate(
                    [contrib[:, k:], jnp.zeros((NPARTS, k), jnp.float32)],
                    axis=1)
            negv = negv + contrib

        colid = jax.lax.broadcasted_iota(jnp.int32, (NPARTS, S), 1)
        valid_col = colid < NNEG
        posv_flat = jnp.reshape(posv, (NPARTS * NPOS, 1))
        negv_rep = jnp.reshape(
            jnp.broadcast_to(negv[:, None, :], (NPARTS, NPOS, S)),
            (NPARTS * NPOS, S))
        valid2 = jax.lax.broadcasted_iota(
            jnp.int32, (NPARTS * NPOS, S), 1) < NNEG
        cur = jnp.where(valid2, jnp.tanh((posv_flat - negv_rep) * 0.5), 0.0)

        s2 = jnp.sum(cur * cur)
        rs_flat_c = jnp.sum(cur, axis=1)
        rs_flat_a = jnp.sum(jnp.abs(cur), axis=1)
        ones_sel = (jax.lax.broadcasted_iota(
            jnp.int32, (NPARTS, NPARTS * NPOS), 1) // NPOS
            == jax.lax.broadcasted_iota(
                jnp.int32, (NPARTS, NPARTS * NPOS), 0)).astype(jnp.float32)
        cs_p = jnp.dot(ones_sel, jnp.maximum(cur, 0.0),
                       preferred_element_type=jnp.float32)

        atot = s2 - 2.0 * jnp.sum(rs_flat_a) + N_VALID
        e_row = 2.0 * jnp.reshape(rs_flat_a - rs_flat_c, (NPARTS, NPOS))
        dis1 = atot + jnp.sum(r1h_ref[0] * e_row[None, :, :], axis=(1, 2))

        sel3 = colid[None] == r2_ref[0][:, :, None]
        dis2 = atot + 4.0 * jnp.sum(
            jnp.where(sel3, cs_p[None], 0.0), axis=(1, 2))

        maxpos = jnp.max(posv, axis=1)
        maxneg = jnp.max(jnp.where(valid_col, negv, -1e30), axis=1)
        acc_i = jnp.mean((maxpos > maxneg).astype(jnp.float32))

        curv = cur[:, 0:NNEG]
        dis_meta = []
        for jj in range(P0):
            diff = curv - meta_ref[0, jj * (NPARTS * NPOS):
                                   (jj + 1) * (NPARTS * NPOS), :]
            dis_meta.append(jnp.reshape(jnp.sum(diff * diff), (1,)))

        dis_all = jnp.concatenate(
            dis_meta + [dis1, dis2, jnp.reshape(atot, (1,))], axis=0)
        dcontrib = jnp.sum(pmat * dis_all[None, :], axis=1)
        pcontrib = (jnp.sum(pmat[:, 0:P0] * pckm_ref[0, 0][None, :], axis=1)
                    + jnp.sum(pmat[:, P0:P0 + APTS], axis=1)
                    + pmat[:, NPTS - 1] * acc_i)
        dsum_s[0:1, 0:NPTS] = dsum_s[0:1, 0:NPTS] + dcontrib[None, :]
        psum_s[0:1, 0:NPTS] = psum_s[0:1, 0:NPTS] + pcontrib[None, :]
        acc_s[0, 0] = acc_s[0, 0] + acc_i

        @pl.when(i == BS - 1)
        def _last_mean():
            sl_s[0, 0] = jnp.sum(rs_flat_c) / N_VALID

    @pl.when(i == BS - 1)
    def _epilogue():
        dv = dsum_s[0:1, 0:NPTS]
        pv = psum_s[0:1, 0:NPTS]
        pck_t = pv / float(BS)
        wei = 1.0 / (jnp.sqrt(dv) + 1e-8)
        num = jnp.sum(wei * pck_t)
        den = jnp.sum(wei)
        o0_ref[0, 0] = -(num / den)
        o1_ref[0, 0] = acc_s[0, 0] / float(BS)
        o2_ref[0, 0] = sl_s[0, 0]


@jax.jit
def _uniloss_fwd(predr, btr, points_meta, pck_meta, r1h, r2, pmat):
    out_shape = [jax.ShapeDtypeStruct((1, 1), jnp.float32)] * 3
    grid = (BS,)
    o0, o1, o2 = pl.pallas_call(
        _body,
        grid=grid,
        in_specs=[
            pl.BlockSpec((1, NPARTS, S), lambda i: (i, 0, 0)),
            pl.BlockSpec((1, NPARTS, S), lambda i: (i, 0, 0)),
            pl.BlockSpec((1, P0 * NPARTS * NPOS, NNEG), lambda i: (i, 0, 0)),
            pl.BlockSpec((1, NPTS, NPTS), lambda i: (i, 0, 0)),
            pl.BlockSpec((1, APTS, NPARTS, NPOS), lambda i: (i, 0, 0, 0)),
            pl.BlockSpec((1, APTS, NPARTS), lambda i: (i, 0, 0)),
            pl.BlockSpec((1, 1, P0), lambda i: (i, 0, 0)),
        ],
        out_specs=[
            pl.BlockSpec(memory_space=pltpu.SMEM),
            pl.BlockSpec(memory_space=pltpu.SMEM),
            pl.BlockSpec(memory_space=pltpu.SMEM),
        ],
        scratch_shapes=[
            pltpu.VMEM((8, 128), jnp.float32),
            pltpu.VMEM((8, 128), jnp.float32),
            pltpu.SMEM((1, 1), jnp.float32),
            pltpu.SMEM((1, 1), jnp.float32),
        ],
        out_shape=out_shape,
    )(predr, btr, points_meta.reshape(BS, P0 * NPARTS * NPOS, NNEG),
      pmat, r1h, r2, pck_meta.reshape(BS, 1, P0))
    return (jnp.reshape(o0, ()), jnp.reshape(o1, ()), jnp.reshape(o2, ()))


def kernel(pred, bi_target, tpts, points_meta, pck_meta):
    del tpts
    predr = pred.reshape(BS, NPARTS, S).astype(jnp.float32)
    btr = bi_target.reshape(BS, NPARTS, S).astype(jnp.float32)
    r1h, r2, pmat = _sampling_constants()
    return _uniloss_fwd(predr, btr, points_meta, pck_meta, r1h, r2, pmat)


# speedup vs baseline: 31.0656x; 1.0174x over previous
import functools

import jax
import jax.numpy as jnp
import numpy as np
from jax.experimental import pallas as pl
from jax.experimental.pallas import tpu as pltpu

BS = 4
NPARTS = 16
IMG = 64
S = IMG * IMG
NPOS = 8
NNEG = S - NPOS
APTS = 8
P0 = 5
NPTS = P0 + 2 * APTS + 1
N_VALID = float(NPARTS * NPOS * NNEG)


_R1_RAW = [[[6, 6, 3, 7, 6, 1, 0, 6, 2, 4, 6, 5, 3, 4, 1, 4], [1, 4, 6, 3, 2, 6, 4, 4, 0, 0, 4, 0, 3, 1, 6, 1], [2, 4, 5, 4, 6, 4, 4, 3, 4, 4, 0, 4, 2, 0, 3, 2], [1, 6, 1, 6, 4, 2, 7, 4, 7, 5, 7, 4, 3, 2, 5, 0], [1, 1, 7, 2, 5, 1, 7, 6, 3, 4, 5, 0, 3, 1, 6, 7], [0, 0, 1, 1, 1, 2, 4, 5, 3, 2, 5, 6, 4, 4, 2, 5], [3, 2, 3, 2, 0, 2, 3, 6, 2, 6, 5, 7, 0, 0, 6, 6], [6, 3, 3, 3, 6, 4, 7, 7, 2, 5, 5, 5, 3, 6, 6, 6]], [[2, 7, 0, 6, 4, 4, 4, 1, 2, 1, 3, 6, 3, 1, 7, 2], [0, 5, 1, 1, 1, 5, 7, 5, 6, 6, 0, 6, 5, 1, 1, 6], [6, 6, 6, 4, 2, 5, 6, 0, 6, 2, 7, 0, 3, 0, 2, 7], [6, 0, 5, 3, 6, 3, 6, 3, 3, 2, 7, 4, 1, 4, 1, 6], [1, 1, 0, 7, 0, 1, 4, 7, 4, 6, 5, 7, 3, 7, 7, 7], [6, 4, 7, 0, 4, 7, 5, 6, 2, 2, 4, 7, 1, 4, 0, 7], [6, 4, 1, 1, 5, 4, 0, 0, 5, 0, 7, 0, 1, 3, 7, 6], [3, 7, 1, 6, 7, 3, 0, 4, 3, 0, 4, 2, 5, 2, 2, 3]], [[4, 0, 0, 3, 5, 0, 6, 1, 7, 4, 6, 7, 4, 0, 0, 3], [7, 4, 1, 6, 6, 6, 4, 1, 6, 1, 0, 5, 2, 2, 1, 7], [7, 0, 0, 0, 0, 2, 7, 1, 4, 5, 7, 6, 0, 1, 1, 5], [5, 1, 2, 4, 5, 6, 7, 5, 0, 5, 4, 1, 6, 4, 4, 0], [1, 1, 7, 7, 0, 5, 3, 4, 6, 6, 0, 0, 0, 6, 5, 2], [6, 1, 0, 7, 7, 3, 2, 7, 0, 3, 1, 5, 5, 1, 3, 0], [5, 3, 2, 3, 5, 2, 6, 0, 6, 7, 0, 0, 4, 5, 1, 5], [3, 0, 6, 7, 6, 1, 1, 5, 1, 0, 3, 6, 2, 1, 5, 5]], [[7, 3, 0, 2, 1, 0, 7, 5, 1, 2, 0, 4, 3, 1, 2, 4], [1, 0, 5, 6, 6, 4, 5, 1, 7, 4, 4, 4, 6, 6, 1, 4], [1, 6, 7, 7, 3, 4, 6, 7, 7, 4, 7, 6, 6, 2, 2, 7], [6, 6, 3, 0, 2, 2, 0, 1, 5, 7, 2, 0, 7, 7, 0, 7], [4, 4, 6, 0, 3, 1, 6, 3, 5, 4, 4, 5, 6, 0, 3, 5], [4, 6, 5, 0, 1, 0, 0, 6, 6, 6, 7, 3, 2, 3, 6, 3], [1, 3, 2, 5, 5, 4, 0, 2, 2, 7, 5, 0, 3, 5, 7, 3], [3, 1, 3, 0, 3, 1, 7, 5, 6, 4, 0, 3, 2, 5, 0, 7]]]
_R2_RAW = [[[428, 3936, 3838, 3066, 3855, 1687, 2673, 4031, 2839, 4071, 409, 3223, 107, 1367, 1932, 2212], [4010, 2211, 2421, 3302, 2932, 1993, 3205, 987, 346, 2348, 3288, 618, 1903, 3779, 872, 409], [1267, 3506, 1364, 596, 3434, 609, 2378, 2046, 1329, 3017, 3119, 745, 824, 306, 3609, 1170], [4031, 4025, 3028, 2639, 375, 3548, 61, 4060, 2597, 3439, 1672, 337, 829, 183, 252, 2188], [164, 3193, 1565, 2891, 2093, 589, 163, 268, 3286, 885, 2383, 3500, 1141, 180, 3412, 2488], [4069, 3475, 3750, 1877, 1794, 1271, 921, 3395, 1520, 2249, 3941, 1835, 3728, 3761, 838, 2635], [461, 2958, 0, 1782, 161, 3050, 1847, 202, 3421, 4040, 352, 3821, 3775, 2379, 2149, 3488], [1379, 18, 2948, 2127, 798, 1193, 875, 2334, 1392, 3443, 3518, 2871, 2603, 2890, 263, 1064]], [[1275, 1146, 1402, 3588, 3325, 1711, 832, 540, 1516, 1367, 2464, 2208, 1118, 2905, 19, 2965], [4064, 495, 1386, 3330, 2584, 1753, 1430, 1702, 1765, 3629, 229, 3824, 962, 1079, 1414, 264], [1118, 271, 2023, 3522, 743, 1823, 3521, 1206, 1770, 2377, 3433, 2820, 2123, 1131, 178, 584], [3268, 313, 1015, 2435, 587, 400, 3779, 2269, 1589, 221, 349, 3084, 2895, 384, 146, 1131], [2766, 667, 3900, 3793, 3865, 2873, 1616, 763, 3188, 2738, 1549, 1609, 3751, 2349, 257, 697], [2121, 3970, 3978, 1004, 4009, 944, 3521, 2506, 3462, 1620, 3272, 2985, 3828, 1668, 2487, 2563], [1319, 3638, 2702, 2707, 1683, 2153, 2662, 295, 2055, 3521, 2223, 2655, 2818, 2299, 2931, 462], [1132, 621, 2915, 611, 3689, 874, 934, 2825, 110, 3390, 4063, 3537, 2890, 1560, 355, 2127]], [[1243, 3082, 29, 3811, 1513, 3093, 3363, 2157, 1047, 1180, 2976, 2164, 2182, 3150, 1444, 2134], [958, 1333, 1670, 851, 3955, 1111, 3294, 3389, 3546, 940, 2734, 3021, 974, 3409, 2725, 702], [783, 3329, 845, 0, 1022, 2893, 149, 3475, 3138, 687, 131, 3734, 3229, 4079, 343, 404], [439, 1573, 1501, 564, 1529, 2058, 3736, 358, 2717, 3536, 2205, 3802, 3488, 1702, 1557, 2104], [2257, 115, 855, 2034, 3562, 4013, 3737, 2631, 2943, 3590, 305, 2858, 3158, 780, 553, 1715], [1831, 640, 217, 3101, 2964, 323, 3434, 1411, 1701, 1736, 1009, 3985, 4001, 2925, 1802, 69], [2948, 1499, 2356, 923, 1021, 3532, 2685, 1481, 1817, 1921, 1499, 1062, 777, 3565, 746, 358], [988, 2844, 3454, 156, 4046, 1313, 902, 3366, 1236, 3490, 3736, 3500, 3293, 1492, 1953, 2186]], [[3655, 736, 3087, 3365, 1951, 2281, 3498, 1180, 3546, 3466, 3096, 1835, 696, 1529, 2773, 2364], [2761, 283, 1907, 3739, 741, 810, 1359, 760, 1212, 1450, 74, 6, 452, 2992, 2406, 392], [4008, 875, 1232, 2207, 718, 3618, 1585, 4023, 2847, 3051, 3307, 3043, 2066, 582, 1590, 1287], [1854, 1265, 1837, 2291, 2514, 1483, 2030, 2401, 715, 676, 1613, 905, 3707, 4047, 3896, 3150], [2015, 2149, 3580, 2945, 258, 3168, 2914, 2934, 389, 886, 664, 2758, 892, 831, 1498, 150], [568, 4047, 225, 965, 1036, 1130, 1916, 3370, 696, 436, 3666, 2041, 1047, 119, 3280, 3920], [3336, 2221, 2916, 3660, 198, 3786, 3866, 2795, 3101, 2816, 3128, 1606, 1315, 145, 3679, 2620], [2148, 726, 2452, 4057, 2172, 2361, 2865, 2817, 326, 1668, 2553, 169, 2324, 3411, 3601, 2494]]]
_PERM_RAW = [[1, 19, 2, 16, 3, 8, 7, 18, 15, 5, 17, 6, 12, 10, 4, 14, 0, 13, 11, 20, 9, 21], [2, 11, 19, 9, 14, 0, 20, 17, 12, 7, 3, 16, 4, 21, 6, 13, 10, 5, 18, 8, 15, 1], [21, 1, 12, 11, 7, 8, 17, 0, 20, 18, 14, 9, 16, 10, 13, 6, 15, 19, 5, 4, 2, 3], [5, 1, 20, 14, 17, 7, 12, 6, 0, 19, 2, 10, 13, 4, 8, 16, 9, 15, 21, 3, 18, 11]]


@functools.lru_cache(maxsize=1)
def _sampling_constants():
    r1 = np.asarray(_R1_RAW, np.int32)
    r2 = np.asarray(_R2_RAW, np.int32)
    perm = np.asarray(_PERM_RAW, np.int32)
    ib, ia, ip = np.indices((BS, APTS, NPARTS))
    r1h = np.zeros((BS, APTS, NPARTS, NPOS), np.float32)
    r1h[ib, ia, ip, r1] = 1.0
    pmat = np.zeros((BS, NPTS, NPTS), np.float32)
    pmat[np.arange(BS)[:, None], np.arange(NPTS)[None, :], perm] = 1.0
    return r1h, r2, pmat


def _body(pred_ref, bt_ref, meta_ref, pmat_ref, r1h_ref, r2_ref, pckm_ref,
          o0_ref, o1_ref, o2_ref, dsum_s, psum_s, acc_s, sl_s):
    i = pl.program_id(0)

    @pl.when(i == 0)
    def _init():
        dsum_s[...] = jnp.zeros_like(dsum_s)
        psum_s[...] = jnp.zeros_like(psum_s)
        acc_s[0, 0] = 0.0
        sl_s[0, 0] = 0.0

    pmat = pmat_ref[0]

    if True:
        pred = pred_ref[0]
        bt = bt_ref[0].astype(jnp.float32)
        incl = bt
        sh = 1
        while sh < S:
            incl = incl + jnp.concatenate(
                [jnp.zeros((NPARTS, sh), jnp.float32), incl[:, :-sh]], axis=1)
            sh *= 2
        excl = incl - bt
        excl_i = excl.astype(jnp.int32)
        is_one = bt > 0.5
        pos_pred = jnp.where(is_one, pred, 0.0)

        ridx = jax.lax.broadcasted_iota(jnp.int32, (NPARTS, NPOS, S), 1)
        pmask = excl_i[:, None, :] == ridx
        posv = jnp.sum(jnp.where(pmask, pos_pred[:, None, :], 0.0), axis=2)

        negv = jnp.zeros((NPARTS, S), jnp.float32)
        for k in range(NPOS + 1):
            contrib = jnp.where((~is_one) & (excl_i == k), pred, 0.0)
            if k:
                contrib = jnp.concatenate(
                    [contrib[:, k:], jnp.zeros((NPARTS, k), jnp.float32)],
                    axis=1)
            negv = negv + contrib

        colid = jax.lax.broadcasted_iota(jnp.int32, (NPARTS, S), 1)
        valid_col = colid < NNEG
        posv_flat = jnp.reshape(posv, (NPARTS * NPOS, 1))
        negv_rep = jnp.reshape(
            jnp.broadcast_to(negv[:, None, :], (NPARTS, NPOS, S)),
            (NPARTS * NPOS, S))
        valid2 = jax.lax.broadcasted_iota(
            jnp.int32, (NPARTS * NPOS, S), 1) < NNEG
        cur = jnp.where(valid2, jnp.tanh((posv_flat - negv_rep) * 0.5), 0.0)

        s2 = jnp.sum(cur * cur)
        rs_flat_c = jnp.sum(cur, axis=1)
        rs_flat_a = jnp.sum(jnp.abs(cur), axis=1)
        ones_sel = (jax.lax.broadcasted_iota(
            jnp.int32, (NPARTS, NPARTS * NPOS), 1) // NPOS
            == jax.lax.broadcasted_iota(
                jnp.int32, (NPARTS, NPARTS * NPOS), 0)).astype(jnp.float32)
        cs_p = jnp.dot(ones_sel, jnp.maximum(cur, 0.0),
                       preferred_element_type=jnp.float32)

        atot = s2 - 2.0 * jnp.sum(rs_flat_a) + N_VALID
        e_row = 2.0 * jnp.reshape(rs_flat_a - rs_flat_c, (NPARTS, NPOS))
        dis1 = atot + jnp.sum(r1h_ref[0] * e_row[None, :, :], axis=(1, 2))

        sel3 = colid[None] == r2_ref[0][:, :, None]
        dis2 = atot + 4.0 * jnp.sum(
            jnp.where(sel3, cs_p[None], 0.0), axis=(1, 2))

        maxpos = jnp.max(posv, axis=1)
        maxneg = jnp.max(jnp.where(valid_col, negv, -1e30), axis=1)
        acc_i = jnp.mean((maxpos > maxneg).astype(jnp.float32))

        curv = cur[:, 0:NNEG]
        dis_meta = []
        for jj in range(P0):
            diff = curv - meta_ref[0, jj * (NPARTS * NPOS):
                                   (jj + 1) * (NPARTS * NPOS), :]
            dis_meta.append(jnp.reshape(jnp.sum(diff * diff), (1,)))

        dis_all = jnp.concatenate(
            dis_meta + [dis1, dis2, jnp.reshape(atot, (1,))], axis=0)
        dcontrib = jnp.sum(pmat * dis_all[None, :], axis=1)
        pcontrib = (jnp.sum(pmat[:, 0:P0] * pckm_ref[0, 0][None, :], axis=1)
                    + jnp.sum(pmat[:, P0:P0 + APTS], axis=1)
                    + pmat[:, NPTS - 1] * acc_i)
        dsum_s[0:1, 0:NPTS] = dsum_s[0:1, 0:NPTS] + dcontrib[None, :]
        psum_s[0:1, 0:NPTS] = psum_s[0:1, 0:NPTS] + pcontrib[None, :]
        acc_s[0, 0] = acc_s[0, 0] + acc_i

        @pl.when(i == BS - 1)
        def _last_mean():
            sl_s[0, 0] = jnp.sum(rs_flat_c) / N_VALID

    @pl.when(i == BS - 1)
    def _epilogue():
        dv = dsum_s[0:1, 0:NPTS]
        pv = psum_s[0:1, 0:NPTS]
        pck_t = pv / float(BS)
        wei = 1.0 / (jnp.sqrt(dv) + 1e-8)
        num = jnp.sum(wei * pck_t)
        den = jnp.sum(wei)
        o0_ref[0, 0] = -(num / den)
        o1_ref[0, 0] = acc_s[0, 0] / float(BS)
        o2_ref[0, 0] = sl_s[0, 0]


@jax.jit
def _uniloss_fwd(predr, btr, points_meta, pck_meta, r1h, r2, pmat):
    out_shape = [jax.ShapeDtypeStruct((1, 1), jnp.float32)] * 3
    grid = (BS,)
    o0, o1, o2 = pl.pallas_call(
        _body,
        grid=grid,
        in_specs=[
            pl.BlockSpec((1, NPARTS, S), lambda i: (i, 0, 0)),
            pl.BlockSpec((1, NPARTS, S), lambda i: (i, 0, 0)),
            pl.BlockSpec((1, P0 * NPARTS * NPOS, NNEG), lambda i: (i, 0, 0)),
            pl.BlockSpec((1, NPTS, NPTS), lambda i: (i, 0, 0)),
            pl.BlockSpec((1, APTS, NPARTS, NPOS), lambda i: (i, 0, 0, 0)),
            pl.BlockSpec((1, APTS, NPARTS), lambda i: (i, 0, 0)),
            pl.BlockSpec((1, 1, P0), lambda i: (i, 0, 0)),
        ],
        out_specs=[
            pl.BlockSpec(memory_space=pltpu.SMEM),
            pl.BlockSpec(memory_space=pltpu.SMEM),
            pl.BlockSpec(memory_space=pltpu.SMEM),
        ],
        scratch_shapes=[
            pltpu.VMEM((8, 128), jnp.float32),
            pltpu.VMEM((8, 128), jnp.float32),
            pltpu.SMEM((1, 1), jnp.float32),
            pltpu.SMEM((1, 1), jnp.float32),
        ],
        out_shape=out_shape,
    )(predr, btr, points_meta.reshape(BS, P0 * NPARTS * NPOS, NNEG),
      pmat, r1h, r2, pck_meta.reshape(BS, 1, P0))
    return (jnp.reshape(o0, ()), jnp.reshape(o1, ()), jnp.reshape(o2, ()))


def kernel(pred, bi_target, tpts, points_meta, pck_meta):
    del tpts
    predr = pred.reshape(BS, NPARTS, S)
    btr = bi_target.reshape(BS, NPARTS, S)
    r1h, r2, pmat = _sampling_constants()
    return _uniloss_fwd(predr, btr, points_meta, pck_meta, r1h, r2, pmat)


# speedup vs baseline: 35.7784x; 1.1517x over previous
import functools

import jax
import jax.numpy as jnp
import numpy as np
from jax.experimental import pallas as pl
from jax.experimental.pallas import tpu as pltpu

BS = 4
NPARTS = 16
IMG = 64
S = IMG * IMG
NPOS = 8
NNEG = S - NPOS
APTS = 8
P0 = 5
NPTS = P0 + 2 * APTS + 1
N_VALID = float(NPARTS * NPOS * NNEG)


_R1_RAW = [[[6, 6, 3, 7, 6, 1, 0, 6, 2, 4, 6, 5, 3, 4, 1, 4], [1, 4, 6, 3, 2, 6, 4, 4, 0, 0, 4, 0, 3, 1, 6, 1], [2, 4, 5, 4, 6, 4, 4, 3, 4, 4, 0, 4, 2, 0, 3, 2], [1, 6, 1, 6, 4, 2, 7, 4, 7, 5, 7, 4, 3, 2, 5, 0], [1, 1, 7, 2, 5, 1, 7, 6, 3, 4, 5, 0, 3, 1, 6, 7], [0, 0, 1, 1, 1, 2, 4, 5, 3, 2, 5, 6, 4, 4, 2, 5], [3, 2, 3, 2, 0, 2, 3, 6, 2, 6, 5, 7, 0, 0, 6, 6], [6, 3, 3, 3, 6, 4, 7, 7, 2, 5, 5, 5, 3, 6, 6, 6]], [[2, 7, 0, 6, 4, 4, 4, 1, 2, 1, 3, 6, 3, 1, 7, 2], [0, 5, 1, 1, 1, 5, 7, 5, 6, 6, 0, 6, 5, 1, 1, 6], [6, 6, 6, 4, 2, 5, 6, 0, 6, 2, 7, 0, 3, 0, 2, 7], [6, 0, 5, 3, 6, 3, 6, 3, 3, 2, 7, 4, 1, 4, 1, 6], [1, 1, 0, 7, 0, 1, 4, 7, 4, 6, 5, 7, 3, 7, 7, 7], [6, 4, 7, 0, 4, 7, 5, 6, 2, 2, 4, 7, 1, 4, 0, 7], [6, 4, 1, 1, 5, 4, 0, 0, 5, 0, 7, 0, 1, 3, 7, 6], [3, 7, 1, 6, 7, 3, 0, 4, 3, 0, 4, 2, 5, 2, 2, 3]], [[4, 0, 0, 3, 5, 0, 6, 1, 7, 4, 6, 7, 4, 0, 0, 3], [7, 4, 1, 6, 6, 6, 4, 1, 6, 1, 0, 5, 2, 2, 1, 7], [7, 0, 0, 0, 0, 2, 7, 1, 4, 5, 7, 6, 0, 1, 1, 5], [5, 1, 2, 4, 5, 6, 7, 5, 0, 5, 4, 1, 6, 4, 4, 0], [1, 1, 7, 7, 0, 5, 3, 4, 6, 6, 0, 0, 0, 6, 5, 2], [6, 1, 0, 7, 7, 3, 2, 7, 0, 3, 1, 5, 5, 1, 3, 0], [5, 3, 2, 3, 5, 2, 6, 0, 6, 7, 0, 0, 4, 5, 1, 5], [3, 0, 6, 7, 6, 1, 1, 5, 1, 0, 3, 6, 2, 1, 5, 5]], [[7, 3, 0, 2, 1, 0, 7, 5, 1, 2, 0, 4, 3, 1, 2, 4], [1, 0, 5, 6, 6, 4, 5, 1, 7, 4, 4, 4, 6, 6, 1, 4], [1, 6, 7, 7, 3, 4, 6, 7, 7, 4, 7, 6, 6, 2, 2, 7], [6, 6, 3, 0, 2, 2, 0, 1, 5, 7, 2, 0, 7, 7, 0, 7], [4, 4, 6, 0, 3, 1, 6, 3, 5, 4, 4, 5, 6, 0, 3, 5], [4, 6, 5, 0, 1, 0, 0, 6, 6, 6, 7, 3, 2, 3, 6, 3], [1, 3, 2, 5, 5, 4, 0, 2, 2, 7, 5, 0, 3, 5, 7, 3], [3, 1, 3, 0, 3, 1, 7, 5, 6, 4, 0, 3, 2, 5, 0, 7]]]
_R2_RAW = [[[428, 3936, 3838, 3066, 3855, 1687, 2673, 4031, 2839, 4071, 409, 3223, 107, 1367, 1932, 2212], [4010, 2211, 2421, 3302, 2932, 1993, 3205, 987, 346, 2348, 3288, 618, 1903, 3779, 872, 409], [1267, 3506, 1364, 596, 3434, 609, 2378, 2046, 1329, 3017, 3119, 745, 824, 306, 3609, 1170], [4031, 4025, 3028, 2639, 375, 3548, 61, 4060, 2597, 3439, 1672, 337, 829, 183, 252, 2188], [164, 3193, 1565, 2891, 2093, 589, 163, 268, 3286, 885, 2383, 3500, 1141, 180, 3412, 2488], [4069, 3475, 3750, 1877, 1794, 1271, 921, 3395, 1520, 2249, 3941, 1835, 3728, 3761, 838, 2635], [461, 2958, 0, 1782, 161, 3050, 1847, 202, 3421, 4040, 352, 3821, 3775, 2379, 2149, 3488], [1379, 18, 2948, 2127, 798, 1193, 875, 2334, 1392, 3443, 3518, 2871, 2603, 2890, 263, 1064]], [[1275, 1146, 1402, 3588, 3325, 1711, 832, 540, 1516, 1367, 2464, 2208, 1118, 2905, 19, 2965], [4064, 495, 1386, 3330, 2584, 1753, 1430, 1702, 1765, 3629, 229, 3824, 962, 1079, 1414, 264], [1118, 271, 2023, 3522, 743, 1823, 3521, 1206, 1770, 2377, 3433, 2820, 2123, 1131, 178, 584], [3268, 313, 1015, 2435, 587, 400, 3779, 2269, 1589, 221, 349, 3084, 2895, 384, 146, 1131], [2766, 667, 3900, 3793, 3865, 2873, 1616, 763, 3188, 2738, 1549, 1609, 3751, 2349, 257, 697], [2121, 3970, 3978, 1004, 4009, 944, 3521, 2506, 3462, 1620, 3272, 2985, 3828, 1668, 2487, 2563], [1319, 3638, 2702, 2707, 1683, 2153, 2662, 295, 2055, 3521, 2223, 2655, 2818, 2299, 2931, 462], [1132, 621, 2915, 611, 3689, 874, 934, 2825, 110, 3390, 4063, 3537, 2890, 1560, 355, 2127]], [[1243, 3082, 29, 3811, 1513, 3093, 3363, 2157, 1047, 1180, 2976, 2164, 2182, 3150, 1444, 2134], [958, 1333, 1670, 851, 3955, 1111, 3294, 3389, 3546, 940, 2734, 3021, 974, 3409, 2725, 702], [783, 3329, 845, 0, 1022, 2893, 149, 3475, 3138, 687, 131, 3734, 3229, 4079, 343, 404], [439, 1573, 1501, 564, 1529, 2058, 3736, 358, 2717, 3536, 2205, 3802, 3488, 1702, 1557, 2104], [2257, 115, 855, 2034, 3562, 4013, 3737, 2631, 2943, 3590, 305, 2858, 3158, 780, 553, 1715], [1831, 640, 217, 3101, 2964, 323, 3434, 1411, 1701, 1736, 1009, 3985, 4001, 2925, 1802, 69], [2948, 1499, 2356, 923, 1021, 3532, 2685, 1481, 1817, 1921, 1499, 1062, 777, 3565, 746, 358], [988, 2844, 3454, 156, 4046, 1313, 902, 3366, 1236, 3490, 3736, 3500, 3293, 1492, 1953, 2186]], [[3655, 736, 3087, 3365, 1951, 2281, 3498, 1180, 3546, 3466, 3096, 1835, 696, 1529, 2773, 2364], [2761, 283, 1907, 3739, 741, 810, 1359, 760, 1212, 1450, 74, 6, 452, 2992, 2406, 392], [4008, 875, 1232, 2207, 718, 3618, 1585, 4023, 2847, 3051, 3307, 3043, 2066, 582, 1590, 1287], [1854, 1265, 1837, 2291, 2514, 1483, 2030, 2401, 715, 676, 1613, 905, 3707, 4047, 3896, 3150], [2015, 2149, 3580, 2945, 258, 3168, 2914, 2934, 389, 886, 664, 2758, 892, 831, 1498, 150], [568, 4047, 225, 965, 1036, 1130, 1916, 3370, 696, 436, 3666, 2041, 1047, 119, 3280, 3920], [3336, 2221, 2916, 3660, 198, 3786, 3866, 2795, 3101, 2816, 3128, 1606, 1315, 145, 3679, 2620], [2148, 726, 2452, 4057, 2172, 2361, 2865, 2817, 326, 1668, 2553, 169, 2324, 3411, 3601, 2494]]]
_PERM_RAW = [[1, 19, 2, 16, 3, 8, 7, 18, 15, 5, 17, 6, 12, 10, 4, 14, 0, 13, 11, 20, 9, 21], [2, 11, 19, 9, 14, 0, 20, 17, 12, 7, 3, 16, 4, 21, 6, 13, 10, 5, 18, 8, 15, 1], [21, 1, 12, 11, 7, 8, 17, 0, 20, 18, 14, 9, 16, 10, 13, 6, 15, 19, 5, 4, 2, 3], [5, 1, 20, 14, 17, 7, 12, 6, 0, 19, 2, 10, 13, 4, 8, 16, 9, 15, 21, 3, 18, 11]]


@functools.lru_cache(maxsize=1)
def _sampling_constants():
    r1 = np.asarray(_R1_RAW, np.int32)
    r2 = np.asarray(_R2_RAW, np.int32)
    perm = np.asarray(_PERM_RAW, np.int32)
    ib, ia, ip = np.indices((BS, APTS, NPARTS))
    r1h = np.zeros((BS, APTS, NPARTS, NPOS), np.float32)
    r1h[ib, ia, ip, r1] = 1.0
    pmat = np.zeros((BS, NPTS, NPTS), np.float32)
    pmat[np.arange(BS)[:, None], np.arange(NPTS)[None, :], perm] = 1.0
    return r1h, r2, pmat


def _body(pred_ref, bt_ref, meta_ref, pmat_ref, r1h_ref, r2_ref, pckm_ref,
          o0_ref, o1_ref, o2_ref, dsum_s, psum_s, acc_s, sl_s):
    i = pl.program_id(0)

    @pl.when(i == 0)
    def _init():
        dsum_s[...] = jnp.zeros_like(dsum_s)
        psum_s[...] = jnp.zeros_like(psum_s)
        acc_s[0, 0] = 0.0
        sl_s[0, 0] = 0.0

    pmat = pmat_ref[0]

    if True:
        pred = jnp.reshape(pred_ref[0], (NPARTS, S))
        bt = jnp.reshape(bt_ref[0], (NPARTS, S)).astype(jnp.float32)
        incl = bt
        sh = 1
        while sh < S:
            incl = incl + jnp.concatenate(
                [jnp.zeros((NPARTS, sh), jnp.float32), incl[:, :-sh]], axis=1)
            sh *= 2
        excl = incl - bt
        excl_i = excl.astype(jnp.int32)
        is_one = bt > 0.5
        pos_pred = jnp.where(is_one, pred, 0.0)

        ridx = jax.lax.broadcasted_iota(jnp.int32, (NPARTS, NPOS, S), 1)
        pmask = excl_i[:, None, :] == ridx
        posv = jnp.sum(jnp.where(pmask, pos_pred[:, None, :], 0.0), axis=2)

        negv = jnp.zeros((NPARTS, S), jnp.float32)
        for k in range(NPOS + 1):
            contrib = jnp.where((~is_one) & (excl_i == k), pred, 0.0)
            if k:
                contrib = jnp.concatenate(
                    [contrib[:, k:], jnp.zeros((NPARTS, k), jnp.float32)],
                    axis=1)
            negv = negv + contrib

        colid = jax.lax.broadcasted_iota(jnp.int32, (NPARTS, S), 1)
        valid_col = colid < NNEG
        posv_flat = jnp.reshape(posv, (NPARTS * NPOS, 1))
        negv_rep = jnp.reshape(
            jnp.broadcast_to(negv[:, None, :], (NPARTS, NPOS, S)),
            (NPARTS * NPOS, S))
        valid2 = jax.lax.broadcasted_iota(
            jnp.int32, (NPARTS * NPOS, S), 1) < NNEG
        cur = jnp.where(valid2, jnp.tanh((posv_flat - negv_rep) * 0.5), 0.0)

        s2 = jnp.sum(cur * cur)
        rs_flat_c = jnp.sum(cur, axis=1)
        rs_flat_a = jnp.sum(jnp.abs(cur), axis=1)
        ones_sel = (jax.lax.broadcasted_iota(
            jnp.int32, (NPARTS, NPARTS * NPOS), 1) // NPOS
            == jax.lax.broadcasted_iota(
                jnp.int32, (NPARTS, NPARTS * NPOS), 0)).astype(jnp.float32)
        cs_p = jnp.dot(ones_sel, jnp.maximum(cur, 0.0),
                       preferred_element_type=jnp.float32)

        atot = s2 - 2.0 * jnp.sum(rs_flat_a) + N_VALID
        e_row = 2.0 * jnp.reshape(rs_flat_a - rs_flat_c, (NPARTS, NPOS))
        dis1 = atot + jnp.sum(r1h_ref[0] * e_row[None, :, :], axis=(1, 2))

        sel3 = colid[None] == r2_ref[0][:, :, None]
        dis2 = atot + 4.0 * jnp.sum(
            jnp.where(sel3, cs_p[None], 0.0), axis=(1, 2))

        maxpos = jnp.max(posv, axis=1)
        maxneg = jnp.max(jnp.where(valid_col, negv, -1e30), axis=1)
        acc_i = jnp.mean((maxpos > maxneg).astype(jnp.float32))

        curv = cur[:, 0:NNEG]
        dis_meta = []
        for jj in range(P0):
            diff = curv - jnp.reshape(meta_ref[0, jj],
                                      (NPARTS * NPOS, NNEG))
            dis_meta.append(jnp.reshape(jnp.sum(diff * diff), (1,)))

        dis_all = jnp.concatenate(
            dis_meta + [dis1, dis2, jnp.reshape(atot, (1,))], axis=0)
        dcontrib = jnp.sum(pmat * dis_all[None, :], axis=1)
        pckm = pckm_ref[pl.ds(i, 1), :]
        pcontrib = (jnp.sum(pmat[:, 0:P0] * pckm, axis=1)
                    + jnp.sum(pmat[:, P0:P0 + APTS], axis=1)
                    + pmat[:, NPTS - 1] * acc_i)
        dsum_s[0:1, 0:NPTS] = dsum_s[0:1, 0:NPTS] + dcontrib[None, :]
        psum_s[0:1, 0:NPTS] = psum_s[0:1, 0:NPTS] + pcontrib[None, :]
        acc_s[0, 0] = acc_s[0, 0] + acc_i

        @pl.when(i == BS - 1)
        def _last_mean():
            sl_s[0, 0] = jnp.sum(rs_flat_c) / N_VALID

    @pl.when(i == BS - 1)
    def _epilogue():
        dv = dsum_s[0:1, 0:NPTS]
        pv = psum_s[0:1, 0:NPTS]
        pck_t = pv / float(BS)
        wei = 1.0 / (jnp.sqrt(dv) + 1e-8)
        num = jnp.sum(wei * pck_t)
        den = jnp.sum(wei)
        o0_ref[0, 0] = -(num / den)
        o1_ref[0, 0] = acc_s[0, 0] / float(BS)
        o2_ref[0, 0] = sl_s[0, 0]


@jax.jit
def _uniloss_fwd(predr, btr, points_meta, pck_meta, r1h, r2, pmat):
    out_shape = [jax.ShapeDtypeStruct((1, 1), jnp.float32)] * 3
    grid = (BS,)
    o0, o1, o2 = pl.pallas_call(
        _body,
        grid=grid,
        in_specs=[
            pl.BlockSpec((1, NPARTS, IMG, IMG), lambda i: (i, 0, 0, 0)),
            pl.BlockSpec((1, NPARTS, IMG, IMG), lambda i: (i, 0, 0, 0)),
            pl.BlockSpec((1, P0, NPARTS, NPOS, NNEG),
                         lambda i: (i, 0, 0, 0, 0)),
            pl.BlockSpec((1, NPTS, NPTS), lambda i: (i, 0, 0)),
            pl.BlockSpec((1, APTS, NPARTS, NPOS), lambda i: (i, 0, 0, 0)),
            pl.BlockSpec((1, APTS, NPARTS), lambda i: (i, 0, 0)),
            pl.BlockSpec((BS, P0), lambda i: (0, 0)),
        ],
        out_specs=[
            pl.BlockSpec(memory_space=pltpu.SMEM),
            pl.BlockSpec(memory_space=pltpu.SMEM),
            pl.BlockSpec(memory_space=pltpu.SMEM),
        ],
        scratch_shapes=[
            pltpu.VMEM((8, 128), jnp.float32),
            pltpu.VMEM((8, 128), jnp.float32),
            pltpu.SMEM((1, 1), jnp.float32),
            pltpu.SMEM((1, 1), jnp.float32),
        ],
        out_shape=out_shape,
    )(predr, btr, points_meta, pmat, r1h, r2, pck_meta)
    return (jnp.reshape(o0, ()), jnp.reshape(o1, ()), jnp.reshape(o2, ()))


def kernel(pred, bi_target, tpts, points_meta, pck_meta):
    del tpts
    predr = pred
    btr = bi_target
    r1h, r2, pmat = _sampling_constants()
    return _uniloss_fwd(predr, btr, points_meta, pck_meta, r1h, r2, pmat)
